# baseline (speedup 1.0000x reference)
import functools

import jax
import jax.numpy as jnp
from jax import lax
from jax.experimental import pallas as pl
from jax.experimental.pallas import tpu as pltpu
from jax.experimental.pallas import tpu_sc as plsc

B = 16384
EM = 32
EG = 8

NC = 2
NS = 16
NW = NC * NS
BPW = B // NW
CH = 128
NCH = BPW // CH

BB = 2048


def _gather_body(user_h, item_h, uem_h, iem_h, ueg_h, ieg_h,
                 o_um, o_im, o_ug, o_ig,
                 uidx, iidx, bum, bim, bug, big, sem):
    wid = lax.axis_index("s") * NC + lax.axis_index("c")
    base = wid * BPW
    pltpu.sync_copy(user_h.at[wid], uidx)
    pltpu.sync_copy(item_h.at[wid], iidx)
    copies = []
    for j in range(NCH):
        dst = pl.ds(j * CH, CH)
        copies.append(pltpu.async_copy(uem_h.at[uidx.at[j]], bum.at[dst], sem))
        copies.append(pltpu.async_copy(iem_h.at[iidx.at[j]], bim.at[dst], sem))
        copies.append(pltpu.async_copy(ueg_h.at[uidx.at[j]], bug.at[dst], sem))
        copies.append(pltpu.async_copy(ieg_h.at[iidx.at[j]], big.at[dst], sem))
    for c in copies:
        c.wait()
    row = pl.ds(base, BPW)
    pltpu.sync_copy(bum, o_um.at[row])
    pltpu.sync_copy(bim, o_im.at[row])
    pltpu.sync_copy(bug, o_ug.at[row])
    pltpu.sync_copy(big, o_ig.at[row])


@functools.lru_cache(maxsize=1)
def _sc_gather():
    return pl.kernel(
        _gather_body,
        out_type=[
            jax.ShapeDtypeStruct((B, EM), jnp.float32),
            jax.ShapeDtypeStruct((B, EM), jnp.float32),
            jax.ShapeDtypeStruct((B, EG), jnp.float32),
            jax.ShapeDtypeStruct((B, EG), jnp.float32),
        ],
        mesh=plsc.VectorSubcoreMesh(
            core_axis_name="c", subcore_axis_name="s",
            num_cores=NC, num_subcores=NS),
        scratch_types=[
            pltpu.VMEM((NCH, CH), jnp.int32),
            pltpu.VMEM((NCH, CH), jnp.int32),
            pltpu.VMEM((BPW, EM), jnp.float32),
            pltpu.VMEM((BPW, EM), jnp.float32),
            pltpu.VMEM((BPW, EG), jnp.float32),
            pltpu.VMEM((BPW, EG), jnp.float32),
            pltpu.SemaphoreType.DMA,
        ],
        compiler_params=pltpu.CompilerParams(use_tc_tiling_on_sc=False),
    )


def _dense_body(um, im, ug, ig, w1u, w1i, b1, w2, b2, w3, b3, w4, b4,
                wp, bp, out):
    f32 = jnp.float32
    h = (jnp.dot(um[:], w1u[:], preferred_element_type=f32)
         + jnp.dot(im[:], w1i[:], preferred_element_type=f32) + b1[:])
    h = jnp.maximum(h, 0.0)
    h = jnp.maximum(jnp.dot(h, w2[:], preferred_element_type=f32) + b2[:], 0.0)
    h = jnp.maximum(jnp.dot(h, w3[:], preferred_element_type=f32) + b3[:], 0.0)
    h = jnp.maximum(jnp.dot(h, w4[:], preferred_element_type=f32) + b4[:], 0.0)
    g = ug[:] * ig[:]
    comb = jnp.concatenate([g, h], axis=1)
    z = jnp.sum(comb * wp[:], axis=1) + bp[0]
    out[:] = jax.nn.sigmoid(z)


def _make_dense():
    full = lambda r, c: pl.BlockSpec((r, c), lambda i: (0, 0))
    return pl.pallas_call(
        _dense_body,
        grid=(B // BB,),
        in_specs=[
            pl.BlockSpec((BB, EM), lambda i: (i, 0)),
            pl.BlockSpec((BB, EM), lambda i: (i, 0)),
            pl.BlockSpec((BB, EG), lambda i: (i, 0)),
            pl.BlockSpec((BB, EG), lambda i: (i, 0)),
            full(EM, 64),
            full(EM, 64),
            full(1, 64),
            full(64, 32),
            full(1, 32),
            full(32, 16),
            full(1, 16),
            full(16, 8),
            full(1, 8),
            full(1, EG + 8),
            pl.BlockSpec((1,), lambda i: (0,)),
        ],
        out_specs=pl.BlockSpec((BB,), lambda i: (i,)),
        out_shape=jax.ShapeDtypeStruct((B,), jnp.float32),
    )


_dense = _make_dense()


def kernel(user, item, UE_mlp, IE_mlp, UE_gmf, IE_gmf,
           W1, b1, W2, b2, W3, b3, W4, b4, Wp, bp):
    user_r = user.astype(jnp.int32).reshape(NW, NCH, CH)
    item_r = item.astype(jnp.int32).reshape(NW, NCH, CH)
    um, im, ug, ig = _sc_gather()(user_r, item_r, UE_mlp, IE_mlp,
                                  UE_gmf, IE_gmf)
    return _dense(um, im, ug, ig,
                  W1[:EM], W1[EM:], b1.reshape(1, -1),
                  W2, b2.reshape(1, -1),
                  W3, b3.reshape(1, -1),
                  W4, b4.reshape(1, -1),
                  Wp.reshape(1, -1), bp)


# speedup vs baseline: 4.1556x; 4.1556x over previous
import functools

import jax
import jax.numpy as jnp
from jax import lax
from jax.experimental import pallas as pl
from jax.experimental.pallas import tpu as pltpu
from jax.experimental.pallas import tpu_sc as plsc

B = 16384
EM = 32
EG = 8
NF = 2 * EM + EG

NC = 2
NS = 16
NW = NC * NS
BPW = B // NW
NSL = BPW // 128
CHW = 16


def _gather_body(uw_h, uc_h, iw_h, ic_h, uemT, iemT, uegT, iegT, xt_h,
                 uc_v, ic_v, uw_s, iw_s, wbuf, wbufg, tbuf, sem):
    wid = lax.axis_index("s") * NC + lax.axis_index("c")
    pltpu.sync_copy(uw_h.at[wid], uw_s)
    pltpu.sync_copy(iw_h.at[wid], iw_s)
    pltpu.sync_copy(uc_h.at[wid], uc_v)
    pltpu.sync_copy(ic_h.at[wid], ic_v)
    del uw_h, iw_h, uc_h, ic_h
    iota16 = lax.iota(jnp.int32, 16)

    for sl in range(NSL):
        for tblT, w_s, c_v, fbase in ((uemT, uw_s, uc_v, 0),
                                      (iemT, iw_s, ic_v, EM)):
            def chunk_m(k, carry, tblT=tblT, w_s=w_s, c_v=c_v,
                        fbase=fbase, sl=sl):
                wvec = w_s[sl, pl.ds(k * CHW, CHW)]
                copies = []
                for j in range(CHW):
                    w = wvec[j]
                    off = pl.multiple_of(w * 128, 128)
                    copies.append(pltpu.async_copy(
                        tblT.at[:, pl.ds(off, 128)], wbuf.at[j], sem))
                for c in copies:
                    c.wait()
                c_vec = c_v[sl, pl.ds(k * CHW, CHW)]
                for f in range(EM):
                    v = plsc.load_gather(
                        wbuf, [iota16, jnp.full((16,), f, jnp.int32), c_vec])
                    tbuf[fbase + f, pl.ds(k * CHW, CHW)] = v
                return carry

            lax.fori_loop(0, 128 // CHW, chunk_m, 0)

        for tblT, w_s, c_v, second in ((uegT, uw_s, uc_v, False),
                                       (iegT, iw_s, ic_v, True)):
            def chunk_g(k, carry, tblT=tblT, w_s=w_s, c_v=c_v,
                        second=second, sl=sl):
                wvec = w_s[sl, pl.ds(k * CHW, CHW)]
                copies = []
                for j in range(CHW):
                    w = wvec[j]
                    off = pl.multiple_of(w * 128, 128)
                    copies.append(pltpu.async_copy(
                        tblT.at[:, pl.ds(off, 128)], wbufg.at[j], sem))
                for c in copies:
                    c.wait()
                c_vec = c_v[sl, pl.ds(k * CHW, CHW)]
                for f in range(EG):
                    v = plsc.load_gather(
                        wbufg, [iota16, jnp.full((16,), f, jnp.int32), c_vec])
                    col = pl.ds(k * CHW, CHW)
                    if second:
                        tbuf[2 * EM + f, col] = tbuf[2 * EM + f, col] * v
                    else:
                        tbuf[2 * EM + f, col] = v
                return carry

            lax.fori_loop(0, 128 // CHW, chunk_g, 0)

        pltpu.sync_copy(tbuf, xt_h.at[NSL * wid + sl])


@functools.lru_cache(maxsize=1)
def _sc_gather():
    return pl.kernel(
        _gather_body,
        out_type=jax.ShapeDtypeStruct((B // 128, NF, 128), jnp.float32),
        mesh=plsc.VectorSubcoreMesh(
            core_axis_name="c", subcore_axis_name="s",
            num_cores=NC, num_subcores=NS),
        scratch_types=[
            pltpu.VMEM((NSL, 128), jnp.int32),
            pltpu.VMEM((NSL, 128), jnp.int32),
            pltpu.VMEM((NSL, 128), jnp.int32),
            pltpu.VMEM((NSL, 128), jnp.int32),
            pltpu.VMEM((CHW, EM, 128), jnp.float32),
            pltpu.VMEM((CHW, EG, 128), jnp.float32),
            pltpu.VMEM((NF, 128), jnp.float32),
            pltpu.SemaphoreType.DMA,
        ],
        compiler_params=pltpu.CompilerParams(
            needs_layout_passes=False, disable_bounds_checks=True),
    )


NSB = 16


def _dense_body(xt, w1t, b1, w2t, b2, w3t, b3, w4t, b4, wpg, wph, bp, out):
    f32 = jnp.float32
    for s in range(NSB):
        x = xt[s]
        xm = x[0:2 * EM, :]
        g8 = x[2 * EM:NF, :]
        h = jnp.maximum(jnp.dot(w1t[:], xm, preferred_element_type=f32)
                        + b1[:], 0.0)
        h = jnp.maximum(jnp.dot(w2t[:], h, preferred_element_type=f32)
                        + b2[:], 0.0)
        h = jnp.maximum(jnp.dot(w3t[:], h, preferred_element_type=f32)
                        + b3[:], 0.0)
        h = jnp.maximum(jnp.dot(w4t[:], h, preferred_element_type=f32)
                        + b4[:], 0.0)
        z = (jnp.dot(wph[:], h, preferred_element_type=f32)
             + jnp.dot(wpg[:], g8, preferred_element_type=f32) + bp[:])
        out[s, :] = jax.nn.sigmoid(z)[0]


def _make_dense():
    full = lambda r, c: pl.BlockSpec((r, c), lambda i: (0, 0))
    return pl.pallas_call(
        _dense_body,
        grid=(B // (128 * NSB),),
        in_specs=[
            pl.BlockSpec((NSB, NF, 128), lambda i: (i, 0, 0)),
            full(64, 64),
            full(64, 1),
            full(32, 64),
            full(32, 1),
            full(16, 32),
            full(16, 1),
            full(8, 16),
            full(8, 1),
            full(1, EG),
            full(1, 8),
            full(1, 1),
        ],
        out_specs=pl.BlockSpec((NSB, 128), lambda i: (i, 0)),
        out_shape=jax.ShapeDtypeStruct((B // 128, 128), jnp.float32),
    )


_dense = _make_dense()


def kernel(user, item, UE_mlp, IE_mlp, UE_gmf, IE_gmf,
           W1, b1, W2, b2, W3, b3, W4, b4, Wp, bp):
    user = user.astype(jnp.int32)
    item = item.astype(jnp.int32)
    uw = (user >> 7).reshape(NW, NSL, 128)
    uc = (user & 127).reshape(NW, NSL, 128)
    iw = (item >> 7).reshape(NW, NSL, 128)
    ic = (item & 127).reshape(NW, NSL, 128)
    xt = _sc_gather()(uw, uc, iw, ic,
                      UE_mlp.T, IE_mlp.T, UE_gmf.T, IE_gmf.T)
    out2d = _dense(xt,
                   W1.T, b1.reshape(-1, 1),
                   W2.T, b2.reshape(-1, 1),
                   W3.T, b3.reshape(-1, 1),
                   W4.T, b4.reshape(-1, 1),
                   Wp[:EG].reshape(1, -1), Wp[EG:].reshape(1, -1),
                   bp.reshape(1, 1))
    return out2d.reshape(B)


# speedup vs baseline: 5.2902x; 1.2730x over previous
import functools

import jax
import jax.numpy as jnp
from jax import lax
from jax.experimental import pallas as pl
from jax.experimental.pallas import tpu as pltpu
from jax.experimental.pallas import tpu_sc as plsc

B = 16384
EM = 32
EG = 8
NF = 2 * EM + EG

NC = 2
NS = 16
NW = NC * NS
BPW = B // NW
NSL = BPW // 128
CHW = 16


def _gather_body(uw_h, uc_h, iw_h, ic_h, uemT, iemT, uegT, iegT, xt_h,
                 uc_v, ic_v, uw_s, iw_s, mb0, mb1, gb0, gb1, tbuf, sem):
    wid = lax.axis_index("s") * NC + lax.axis_index("c")
    pltpu.sync_copy(uw_h.at[wid], uw_s)
    pltpu.sync_copy(iw_h.at[wid], iw_s)
    pltpu.sync_copy(uc_h.at[wid], uc_v)
    pltpu.sync_copy(ic_h.at[wid], ic_v)
    del uw_h, iw_h, uc_h, ic_h
    iota16 = lax.iota(jnp.int32, 16)

    def drain(buf):
        for j in range(CHW):
            pltpu.make_async_copy(
                uemT.at[pl.ds(0, buf.shape[1]), pl.ds(0, 128)],
                buf.at[j], sem).wait()

    def slab_body(sl, carry0):
        for tblT, w_s, c_v, fbase in ((uemT, uw_s, uc_v, 0),
                                      (iemT, iw_s, ic_v, EM)):
            def fire_m(k, h, buf, tblT=tblT, w_s=w_s, sl=sl):
                wvec = w_s[sl, pl.ds(k * CHW, CHW)]
                for j in range(CHW):
                    off = pl.multiple_of(wvec[j] * 128, 128)
                    pltpu.async_copy(
                        tblT.at[pl.ds(h * 16, 16), pl.ds(off, 128)],
                        buf.at[j], sem)

            def extract_m(k, h, buf, c_v=c_v, fbase=fbase, sl=sl):
                c_vec = c_v[sl, pl.ds(k * CHW, CHW)]
                for f in range(16):
                    v = plsc.load_gather(
                        buf, [iota16, jnp.full((16,), f, jnp.int32), c_vec])
                    tbuf[fbase + h * 16 + f, pl.ds(k * CHW, CHW)] = v

            fire_m(0, 0, mb0)
            fire_m(0, 1, mb1)

            def body_m(q, carry, fire_m=fire_m, extract_m=extract_m):
                drain(mb0)
                extract_m(q, 0, mb0)
                fire_m(q + 1, 0, mb0)
                drain(mb1)
                extract_m(q, 1, mb1)
                fire_m(q + 1, 1, mb1)
                return carry

            lax.fori_loop(0, 128 // CHW - 1, body_m, 0)
            last = 128 // CHW - 1
            drain(mb0)
            extract_m(last, 0, mb0)
            drain(mb1)
            extract_m(last, 1, mb1)

        for tblT, w_s, c_v, second in ((uegT, uw_s, uc_v, False),
                                       (iegT, iw_s, ic_v, True)):
            def fire_g(k, buf, tblT=tblT, w_s=w_s, sl=sl):
                wvec = w_s[sl, pl.ds(k * CHW, CHW)]
                for j in range(CHW):
                    off = pl.multiple_of(wvec[j] * 128, 128)
                    pltpu.async_copy(
                        tblT.at[:, pl.ds(off, 128)], buf.at[j], sem)

            def extract_g(k, buf, c_v=c_v, second=second, sl=sl):
                c_vec = c_v[sl, pl.ds(k * CHW, CHW)]
                for f in range(EG):
                    v = plsc.load_gather(
                        buf, [iota16, jnp.full((16,), f, jnp.int32), c_vec])
                    col = pl.ds(k * CHW, CHW)
                    if second:
                        tbuf[2 * EM + f, col] = tbuf[2 * EM + f, col] * v
                    else:
                        tbuf[2 * EM + f, col] = v

            fire_g(0, gb0)
            fire_g(1, gb1)

            def body_g(q, carry, fire_g=fire_g, extract_g=extract_g):
                drain(gb0)
                extract_g(2 * q, gb0)
                fire_g(2 * q + 2, gb0)
                drain(gb1)
                extract_g(2 * q + 1, gb1)
                fire_g(2 * q + 3, gb1)
                return carry

            lax.fori_loop(0, 128 // CHW // 2 - 1, body_g, 0)
            last = 128 // CHW - 2
            drain(gb0)
            extract_g(last, gb0)
            drain(gb1)
            extract_g(last + 1, gb1)

        pltpu.sync_copy(tbuf, xt_h.at[NSL * wid + sl])
        return carry0

    lax.fori_loop(0, NSL, slab_body, 0)


@functools.lru_cache(maxsize=1)
def _sc_gather():
    return pl.kernel(
        _gather_body,
        out_type=jax.ShapeDtypeStruct((B // 128, NF, 128), jnp.float32),
        mesh=plsc.VectorSubcoreMesh(
            core_axis_name="c", subcore_axis_name="s",
            num_cores=NC, num_subcores=NS),
        scratch_types=[
            pltpu.VMEM((NSL, 128), jnp.int32),
            pltpu.VMEM((NSL, 128), jnp.int32),
            pltpu.VMEM((NSL, 128), jnp.int32),
            pltpu.VMEM((NSL, 128), jnp.int32),
            pltpu.VMEM((CHW, 16, 128), jnp.float32),
            pltpu.VMEM((CHW, 16, 128), jnp.float32),
            pltpu.VMEM((CHW, EG, 128), jnp.float32),
            pltpu.VMEM((CHW, EG, 128), jnp.float32),
            pltpu.VMEM((NF, 128), jnp.float32),
            pltpu.SemaphoreType.DMA,
        ],
        compiler_params=pltpu.CompilerParams(
            needs_layout_passes=False, disable_bounds_checks=True),
    )


NSB = 16


def _dense_body(xt, w1t, b1, w2t, b2, w3t, b3, w4t, b4, wpg, wph, bp, out):
    f32 = jnp.float32
    for s in range(NSB):
        x = xt[s]
        xm = x[0:2 * EM, :]
        g8 = x[2 * EM:NF, :]
        h = jnp.maximum(jnp.dot(w1t[:], xm, preferred_element_type=f32)
                        + b1[:], 0.0)
        h = jnp.maximum(jnp.dot(w2t[:], h, preferred_element_type=f32)
                        + b2[:], 0.0)
        h = jnp.maximum(jnp.dot(w3t[:], h, preferred_element_type=f32)
                        + b3[:], 0.0)
        h = jnp.maximum(jnp.dot(w4t[:], h, preferred_element_type=f32)
                        + b4[:], 0.0)
        z = (jnp.dot(wph[:], h, preferred_element_type=f32)
             + jnp.dot(wpg[:], g8, preferred_element_type=f32) + bp[:])
        out[s, :] = jax.nn.sigmoid(z)[0]


def _make_dense():
    full = lambda r, c: pl.BlockSpec((r, c), lambda i: (0, 0))
    return pl.pallas_call(
        _dense_body,
        grid=(B // (128 * NSB),),
        in_specs=[
            pl.BlockSpec((NSB, NF, 128), lambda i: (i, 0, 0)),
            full(64, 64),
            full(64, 1),
            full(32, 64),
            full(32, 1),
            full(16, 32),
            full(16, 1),
            full(8, 16),
            full(8, 1),
            full(1, EG),
            full(1, 8),
            full(1, 1),
        ],
        out_specs=pl.BlockSpec((NSB, 128), lambda i: (i, 0)),
        out_shape=jax.ShapeDtypeStruct((B // 128, 128), jnp.float32),
    )


_dense = _make_dense()


def kernel(user, item, UE_mlp, IE_mlp, UE_gmf, IE_gmf,
           W1, b1, W2, b2, W3, b3, W4, b4, Wp, bp):
    user = user.astype(jnp.int32)
    item = item.astype(jnp.int32)
    uw = (user >> 7).reshape(NW, NSL, 128)
    uc = (user & 127).reshape(NW, NSL, 128)
    iw = (item >> 7).reshape(NW, NSL, 128)
    ic = (item & 127).reshape(NW, NSL, 128)
    xt = _sc_gather()(uw, uc, iw, ic,
                      UE_mlp.T, IE_mlp.T, UE_gmf.T, IE_gmf.T)
    out2d = _dense(xt,
                   W1.T, b1.reshape(-1, 1),
                   W2.T, b2.reshape(-1, 1),
                   W3.T, b3.reshape(-1, 1),
                   W4.T, b4.reshape(-1, 1),
                   Wp[:EG].reshape(1, -1), Wp[EG:].reshape(1, -1),
                   bp.reshape(1, 1))
    return out2d.reshape(B)
